# baseline (speedup 1.0000x reference)
import jax
import jax.numpy as jnp
from jax import lax
from jax.experimental import pallas as pl
from jax.experimental.pallas import tpu as pltpu
from jax.experimental.pallas import tpu_sc as plsc

_B, _N, _D, _K = 4, 4096, 256, 512
_ITERS = 10
_CH = 1024
_NC, _NS = 2, 16
_NW = _NC * _NS
_DH = _D // 2
_PR = _N // 4

_DIST_PREC = lax.Precision.DEFAULT
_SEG_PREC = lax.Precision.HIGHEST


def _assign_body(x_ref, c_ref, labels_ref, counts_ref):
    c = c_ref[0]
    b2 = jnp.sum(c * c, axis=1)

    def chunk(nb, counts):
        xc = x_ref[0, pl.ds(nb * _CH, _CH), :]
        ab = lax.dot_general(xc, c, (((1,), (1,)), ((), ())),
                             precision=_DIST_PREC,
                             preferred_element_type=jnp.float32)
        a2 = jnp.sum(xc * xc, axis=1)
        d2 = jnp.maximum(a2[:, None] + b2[None, :] - 2.0 * ab, 0.0)
        m = jnp.min(d2, axis=1)
        kidx = lax.broadcasted_iota(jnp.int32, (_CH, _K), 1)
        lbl = jnp.min(jnp.where(d2 == m[:, None], kidx, _K), axis=1)
        labels_ref[0, 0, pl.ds(nb * _CH, _CH)] = lbl
        onehot_t = (lax.broadcasted_iota(jnp.int32, (_K, _CH), 0)
                    == lbl[None, :]).astype(jnp.float32)
        return counts + jnp.sum(onehot_t, axis=1)

    counts = lax.fori_loop(0, _N // _CH, chunk, jnp.zeros((_K,), jnp.float32))
    counts_ref[0, 0] = counts


def _assign(X, centers):
    return pl.pallas_call(
        _assign_body,
        grid=(_B,),
        in_specs=[
            pl.BlockSpec((1, _N, _D), lambda b: (b, 0, 0)),
            pl.BlockSpec((1, _K, _D), lambda b: (b, 0, 0)),
        ],
        out_specs=[
            pl.BlockSpec((1, 1, _N), lambda b: (b, 0, 0)),
            pl.BlockSpec((1, 1, _K), lambda b: (b, 0, 0)),
        ],
        out_shape=[
            jax.ShapeDtypeStruct((_B, 1, _N), jnp.int32),
            jax.ShapeDtypeStruct((_B, 1, _K), jnp.float32),
        ],
    )(X, centers)


def _segsum_body(x_hbm, labels_hbm, out_hbm, lbl_v, rows_v, acc_v):
    cid = lax.axis_index("c")
    sid = lax.axis_index("s")
    w = sid * _NC + cid
    b = w // 8
    half = (w // 4) % 2
    r = w % 4

    zero16 = jnp.zeros((16,), jnp.float32)

    def zrow(i, _):
        for cc in range(_DH // 16):
            acc_v[i, pl.ds(cc * 16, 16)] = zero16
        return 0

    lax.fori_loop(0, _K, zrow, 0)

    lbase = b * _N + r * _PR
    pltpu.sync_copy(labels_hbm.at[pl.ds(lbase, _PR)], lbl_v)

    xbase = half * (_B * _N) + lbase
    for c8 in range(_PR // 128):
        pltpu.sync_copy(x_hbm.at[pl.ds(xbase + c8 * 128, 128)], rows_v)

        def add_group(g, _):
            lbl16 = lbl_v[pl.ds(c8 * 128 + g * 16, 16)]
            for j in range(16):
                l = lbl16[j]
                for cc in range(_DH // 16):
                    plsc.addupdate(acc_v.at[l, pl.ds(cc * 16, 16)],
                                   rows_v[g * 16 + j, pl.ds(cc * 16, 16)])
            return 0

        lax.fori_loop(0, 8, add_group, 0)

    pltpu.sync_copy(acc_v, out_hbm.at[pl.ds(w * _K, _K)])


_segsum_cache = []


def _segsum(x_split, labels_flat):
    if not _segsum_cache:
        _segsum_cache.append(pl.kernel(
            _segsum_body,
            out_type=jax.ShapeDtypeStruct((_NW * _K, _DH), jnp.float32),
            mesh=plsc.VectorSubcoreMesh(
                core_axis_name="c", subcore_axis_name="s"),
            scratch_types=[
                pltpu.VMEM((_PR,), jnp.int32),
                pltpu.VMEM((128, _DH), jnp.float32),
                pltpu.VMEM((_K, _DH), jnp.float32),
            ],
        ))
    return _segsum_cache[0](x_split, labels_flat)


def _update_body(p_ref, cnt_ref, cprev_ref, lnew_ref, lprev_ref,
                 cntprev_ref, done_ref, cout_ref, lout_ref, cntout_ref,
                 dout_ref):
    done = done_ref[0] == 1
    conv = jnp.bool_(True)
    for b in range(_B):
        cnt = cnt_ref[b, 0]
        for h in range(2):
            base = b * 8 + h * 4
            s = (p_ref[base] + p_ref[base + 1]
                 + p_ref[base + 2] + p_ref[base + 3])
            new_c = s / cnt[:, None]
            c_prev = cprev_ref[b, :, pl.ds(h * _DH, _DH)]
            ok = jnp.abs(c_prev - new_c) <= (1e-8 + 1e-5 * jnp.abs(new_c))
            conv = jnp.logical_and(conv, jnp.all(ok))
            cout_ref[b, :, pl.ds(h * _DH, _DH)] = new_c
    keep = jnp.logical_or(done, conv)

    @pl.when(keep)
    def _():
        cout_ref[...] = cprev_ref[...]

    lout_ref[...] = jnp.where(done, lprev_ref[...], lnew_ref[...])
    cntout_ref[...] = jnp.where(done, cntprev_ref[...], cnt_ref[...])
    dout_ref[0] = keep.astype(jnp.int32)


def _update(partials, counts_new, centers, labels_new, labels, counts, done):
    return pl.pallas_call(
        _update_body,
        in_specs=[
            pl.BlockSpec(memory_space=pltpu.VMEM),
            pl.BlockSpec(memory_space=pltpu.VMEM),
            pl.BlockSpec(memory_space=pltpu.VMEM),
            pl.BlockSpec(memory_space=pltpu.VMEM),
            pl.BlockSpec(memory_space=pltpu.VMEM),
            pl.BlockSpec(memory_space=pltpu.VMEM),
            pl.BlockSpec(memory_space=pltpu.SMEM),
        ],
        out_specs=[
            pl.BlockSpec(memory_space=pltpu.VMEM),
            pl.BlockSpec(memory_space=pltpu.VMEM),
            pl.BlockSpec(memory_space=pltpu.VMEM),
            pl.BlockSpec(memory_space=pltpu.SMEM),
        ],
        out_shape=[
            jax.ShapeDtypeStruct((_B, _K, _D), jnp.float32),
            jax.ShapeDtypeStruct((_B, 1, _N), jnp.int32),
            jax.ShapeDtypeStruct((_B, 1, _K), jnp.float32),
            jax.ShapeDtypeStruct((1,), jnp.int32),
        ],
    )(partials, counts_new, centers, labels_new, labels, counts, done)


def kernel(X):
    B, N, D = X.shape
    perm = jax.random.permutation(jax.random.key(42), N)[:_K]
    centers0 = X[:, perm]
    x_split = jnp.concatenate(
        [X[:, :, :_DH].reshape(-1, _DH), X[:, :, _DH:].reshape(-1, _DH)], 0)

    def body(_, carry):
        centers, labels, counts, done = carry
        labels_new, counts_new = _assign(X, centers)
        partials = _segsum(x_split, labels_new.reshape(B * N))
        partials = partials.reshape(_NW, _K, _DH)
        centers, labels, counts, done = _update(
            partials, counts_new, centers, labels_new, labels, counts, done)
        return centers, labels, counts, done

    init = (centers0,
            jnp.zeros((B, 1, N), jnp.int32),
            jnp.zeros((B, 1, _K), jnp.float32),
            jnp.zeros((1,), jnp.int32))
    centers, labels, counts, _ = lax.fori_loop(0, _ITERS, body, init)
    return centers, labels.reshape(B, N), counts.reshape(B, _K) / float(N)
